# Initial kernel scaffold; baseline (speedup 1.0000x reference)
import jax
import jax.numpy as jnp
from jax.experimental import pallas as pl


def kernel(z, embed, cluster_size):
    raise NotImplementedError("write your pallas kernel here")


# speedup vs baseline: 1.5281x; 1.5281x over previous
import functools
import math

import jax
import jax.numpy as jnp
from jax.experimental import pallas as pl
from jax.experimental.pallas import tpu as pltpu

NUM_CODES = 8192
CODE_DIM = 32
ROW_BLOCK = 256


def _vq_kernel(z_blk, embed, embed_t, cs, zq_out, codes_out, scal_out,
               active_out, soft_acc, cnt_acc, esq, msum, *, n_rows, n_blocks):
    i = pl.program_id(0)

    @pl.when(i == 0)
    def _init():
        esq[...] = jnp.sum(embed_t[...] * embed_t[...], axis=0, keepdims=True)
        soft_acc[...] = jnp.zeros_like(soft_acc)
        cnt_acc[...] = jnp.zeros_like(cnt_acc)
        msum[0, 0] = 0.0

    zb = z_blk[...]
    zsq = jnp.sum(zb * zb, axis=1, keepdims=True)
    prod = jnp.dot(zb, embed_t[...], preferred_element_type=jnp.float32)
    dist = zsq - 2.0 * prod + esq[...]

    rowmin = jnp.min(dist, axis=1, keepdims=True)
    lane = jax.lax.broadcasted_iota(jnp.int32, dist.shape, 1)
    codes = jnp.min(jnp.where(dist == rowmin, lane, NUM_CODES),
                    axis=1, keepdims=True)

    p = jnp.exp(rowmin - dist)
    zsum = jnp.sum(p, axis=1, keepdims=True)
    soft_acc[...] += jnp.sum(p * (1.0 / zsum), axis=0, keepdims=True)

    onehot = jnp.where(codes == lane, 1.0, 0.0)
    cnt_acc[...] += jnp.sum(onehot, axis=0, keepdims=True)
    zq_out[...] = jnp.dot(onehot, embed[...], preferred_element_type=jnp.float32)
    codes_out[...] = codes
    msum[0, 0] += jnp.sum(rowmin)

    @pl.when(i == n_blocks - 1)
    def _finish():
        n = jnp.float32(n_rows)
        avg_soft = soft_acc[...] / n
        usage_entropy = -jnp.sum(avg_soft * jnp.log(avg_soft + 1e-10))
        diversity_loss = -usage_entropy / math.log(NUM_CODES)
        hard_avg = cnt_acc[...] / n
        perplexity = jnp.exp(-jnp.sum(hard_avg * jnp.log(hard_avg + 1e-10)))
        loss = msum[0, 0] / (n * CODE_DIM)
        lane8 = jax.lax.broadcasted_iota(jnp.int32, (1, 8), 1)
        scal_out[...] = (loss * (lane8 <= 1)
                         + diversity_loss * (lane8 == 2)
                         + usage_entropy * (lane8 == 3)
                         + perplexity * (lane8 == 4))
        active = jnp.sum((cs[...] > 1.0).astype(jnp.int32))
        active_out[...] = jnp.broadcast_to(active, (1, 1))


@jax.jit
def kernel(z, embed, cluster_size):
    orig_shape = z.shape
    flat_z = z.reshape(-1, CODE_DIM)
    n_rows = flat_z.shape[0]
    n_blocks = n_rows // ROW_BLOCK
    embed_t = embed.T
    cs = cluster_size.reshape(1, NUM_CODES)

    grid = (n_blocks,)
    out_shapes = (
        jax.ShapeDtypeStruct((n_rows, CODE_DIM), jnp.float32),
        jax.ShapeDtypeStruct((n_rows, 1), jnp.int32),
        jax.ShapeDtypeStruct((1, 8), jnp.float32),
        jax.ShapeDtypeStruct((1, 1), jnp.int32),
    )
    zq, codes, scal, active = pl.pallas_call(
        functools.partial(_vq_kernel, n_rows=n_rows, n_blocks=n_blocks),
        grid=grid,
        in_specs=[
            pl.BlockSpec((ROW_BLOCK, CODE_DIM), lambda i: (i, 0)),
            pl.BlockSpec((NUM_CODES, CODE_DIM), lambda i: (0, 0)),
            pl.BlockSpec((CODE_DIM, NUM_CODES), lambda i: (0, 0)),
            pl.BlockSpec((1, NUM_CODES), lambda i: (0, 0)),
        ],
        out_specs=(
            pl.BlockSpec((ROW_BLOCK, CODE_DIM), lambda i: (i, 0)),
            pl.BlockSpec((ROW_BLOCK, 1), lambda i: (i, 0)),
            pl.BlockSpec((1, 8), lambda i: (0, 0)),
            pl.BlockSpec((1, 1), lambda i: (0, 0)),
        ),
        out_shape=out_shapes,
        scratch_shapes=[
            pltpu.VMEM((1, NUM_CODES), jnp.float32),
            pltpu.VMEM((1, NUM_CODES), jnp.float32),
            pltpu.VMEM((1, NUM_CODES), jnp.float32),
            pltpu.SMEM((1, 1), jnp.float32),
        ],
    )(flat_z, embed, embed_t, cs)

    z_q_out = zq.reshape(orig_shape)
    codes_out = codes.reshape(orig_shape[:-1])
    return (z_q_out, codes_out,
            scal[0, 0], scal[0, 1], scal[0, 2], scal[0, 3], scal[0, 4],
            active[0, 0])


# speedup vs baseline: 1.5415x; 1.0088x over previous
import functools
import math

import jax
import jax.numpy as jnp
from jax.experimental import pallas as pl
from jax.experimental.pallas import tpu as pltpu

NUM_CODES = 8192
CODE_DIM = 32
ROW_BLOCK = 256


def _vq_kernel(z2_blk, embed, embed_t, cs, zq_out, codes_out, scal_out,
               active_out, soft_acc, cnt_acc, esq, msum, *, n_rows, n_blocks):
    i = pl.program_id(0)
    r = z2_blk.shape[0]

    @pl.when(i == 0)
    def _init():
        esq[...] = jnp.sum(embed_t[...] * embed_t[...], axis=0, keepdims=True)
        soft_acc[...] = jnp.zeros_like(soft_acc)
        cnt_acc[...] = jnp.zeros_like(cnt_acc)
        msum[0, 0] = 0.0

    z2 = z2_blk[...]
    zsq = 0.25 * jnp.sum(z2 * z2, axis=1, keepdims=True)
    prod2 = jnp.dot(z2, embed_t[...], preferred_element_type=jnp.float32)
    dist = (zsq - prod2) + esq[...]

    rowmin = jnp.min(dist, axis=1, keepdims=True)
    lane = jax.lax.broadcasted_iota(jnp.int32, dist.shape, 1)
    codes = jnp.min(jnp.where(dist == rowmin, lane, NUM_CODES),
                    axis=1, keepdims=True)

    p = jnp.exp(rowmin - dist)
    ones_k = jnp.full((dist.shape[1], 1), 1.0, jnp.float32)
    zsum = jnp.dot(p, ones_k, preferred_element_type=jnp.float32)
    invz_row = jnp.transpose(1.0 / zsum)
    soft_acc[...] += jnp.dot(invz_row, p, preferred_element_type=jnp.float32)

    onehot = jnp.where(codes == lane, 1.0, 0.0)
    ones_r = jnp.full((1, r), 1.0, jnp.float32)
    cnt_acc[...] += jnp.dot(ones_r, onehot, preferred_element_type=jnp.float32)
    zq_out[...] = jnp.dot(onehot, embed[...], preferred_element_type=jnp.float32)
    codes_out[...] = codes
    msum[0, 0] += jnp.sum(rowmin)

    @pl.when(i == n_blocks - 1)
    def _finish():
        n = jnp.float32(n_rows)
        avg_soft = soft_acc[...] / n
        usage_entropy = -jnp.sum(avg_soft * jnp.log(avg_soft + 1e-10))
        diversity_loss = -usage_entropy / math.log(NUM_CODES)
        hard_avg = cnt_acc[...] / n
        perplexity = jnp.exp(-jnp.sum(hard_avg * jnp.log(hard_avg + 1e-10)))
        loss = msum[0, 0] / (n * CODE_DIM)
        lane8 = jax.lax.broadcasted_iota(jnp.int32, (1, 8), 1)
        scal_out[...] = (loss * (lane8 <= 1)
                         + diversity_loss * (lane8 == 2)
                         + usage_entropy * (lane8 == 3)
                         + perplexity * (lane8 == 4))
        active = jnp.sum((cs[...] > 1.0).astype(jnp.int32))
        active_out[...] = jnp.broadcast_to(active, (1, 1))


@jax.jit
def kernel(z, embed, cluster_size):
    orig_shape = z.shape
    flat_z = z.reshape(-1, CODE_DIM)
    n_rows = flat_z.shape[0]
    n_blocks = n_rows // ROW_BLOCK
    embed_t = embed.T
    cs = cluster_size.reshape(1, NUM_CODES)

    grid = (n_blocks,)
    out_shapes = (
        jax.ShapeDtypeStruct((n_rows, CODE_DIM), jnp.float32),
        jax.ShapeDtypeStruct((n_rows, 1), jnp.int32),
        jax.ShapeDtypeStruct((1, 8), jnp.float32),
        jax.ShapeDtypeStruct((1, 1), jnp.int32),
    )
    zq, codes, scal, active = pl.pallas_call(
        functools.partial(_vq_kernel, n_rows=n_rows, n_blocks=n_blocks),
        grid=grid,
        in_specs=[
            pl.BlockSpec((ROW_BLOCK, CODE_DIM), lambda i: (i, 0)),
            pl.BlockSpec((NUM_CODES, CODE_DIM), lambda i: (0, 0)),
            pl.BlockSpec((CODE_DIM, NUM_CODES), lambda i: (0, 0)),
            pl.BlockSpec((1, NUM_CODES), lambda i: (0, 0)),
        ],
        out_specs=(
            pl.BlockSpec((ROW_BLOCK, CODE_DIM), lambda i: (i, 0)),
            pl.BlockSpec((ROW_BLOCK, 1), lambda i: (i, 0)),
            pl.BlockSpec((1, 8), lambda i: (0, 0)),
            pl.BlockSpec((1, 1), lambda i: (0, 0)),
        ),
        out_shape=out_shapes,
        scratch_shapes=[
            pltpu.VMEM((1, NUM_CODES), jnp.float32),
            pltpu.VMEM((1, NUM_CODES), jnp.float32),
            pltpu.VMEM((1, NUM_CODES), jnp.float32),
            pltpu.SMEM((1, 1), jnp.float32),
        ],
    )(2.0 * flat_z, embed, embed_t, cs)

    z_q_out = zq.reshape(orig_shape)
    codes_out = codes.reshape(orig_shape[:-1])
    return (z_q_out, codes_out,
            scal[0, 0], scal[0, 1], scal[0, 2], scal[0, 3], scal[0, 4],
            active[0, 0])


# speedup vs baseline: 1.5881x; 1.0303x over previous
import functools
import math

import jax
import jax.numpy as jnp
from jax.experimental import pallas as pl
from jax.experimental.pallas import tpu as pltpu

NUM_CODES = 8192
CODE_DIM = 32
ROW_BLOCK = 512


def _vq_kernel(z2_blk, embed, embed_t, cs, zq_out, codes_out, scal_out,
               active_out, soft_acc, cnt_acc, esq, msum, *, n_rows, n_blocks):
    i = pl.program_id(0)
    r = z2_blk.shape[0]

    @pl.when(i == 0)
    def _init():
        esq[...] = jnp.sum(embed_t[...] * embed_t[...], axis=0, keepdims=True)
        soft_acc[...] = jnp.zeros_like(soft_acc)
        cnt_acc[...] = jnp.zeros_like(cnt_acc)
        msum[0, 0] = 0.0

    z2 = z2_blk[...]
    zsq = 0.25 * jnp.sum(z2 * z2, axis=1, keepdims=True)
    prod2 = jnp.dot(z2, embed_t[...], preferred_element_type=jnp.float32)
    dist = (zsq - prod2) + esq[...]

    rowmin = jnp.min(dist, axis=1, keepdims=True)
    lane = jax.lax.broadcasted_iota(jnp.int32, dist.shape, 1)
    codes = jnp.min(jnp.where(dist == rowmin, lane, NUM_CODES),
                    axis=1, keepdims=True)

    p = jnp.exp(rowmin - dist)
    ones_k = jnp.full((dist.shape[1], 1), 1.0, jnp.float32)
    zsum = jnp.dot(p, ones_k, preferred_element_type=jnp.float32)
    invz_row = jnp.transpose(1.0 / zsum)
    soft_acc[...] += jnp.dot(invz_row, p, preferred_element_type=jnp.float32)

    onehot = jnp.where(codes == lane, 1.0, 0.0)
    ones_r = jnp.full((1, r), 1.0, jnp.float32)
    cnt_acc[...] += jnp.dot(ones_r, onehot, preferred_element_type=jnp.float32)
    zq_out[...] = jnp.dot(onehot, embed[...], preferred_element_type=jnp.float32)
    codes_out[...] = codes
    msum[0, 0] += jnp.sum(rowmin)

    @pl.when(i == n_blocks - 1)
    def _finish():
        n = jnp.float32(n_rows)
        avg_soft = soft_acc[...] / n
        usage_entropy = -jnp.sum(avg_soft * jnp.log(avg_soft + 1e-10))
        diversity_loss = -usage_entropy / math.log(NUM_CODES)
        hard_avg = cnt_acc[...] / n
        perplexity = jnp.exp(-jnp.sum(hard_avg * jnp.log(hard_avg + 1e-10)))
        loss = msum[0, 0] / (n * CODE_DIM)
        lane8 = jax.lax.broadcasted_iota(jnp.int32, (1, 8), 1)
        scal_out[...] = (loss * (lane8 <= 1)
                         + diversity_loss * (lane8 == 2)
                         + usage_entropy * (lane8 == 3)
                         + perplexity * (lane8 == 4))
        active = jnp.sum((cs[...] > 1.0).astype(jnp.int32))
        active_out[...] = jnp.broadcast_to(active, (1, 1))


@jax.jit
def kernel(z, embed, cluster_size):
    orig_shape = z.shape
    flat_z = z.reshape(-1, CODE_DIM)
    n_rows = flat_z.shape[0]
    n_blocks = n_rows // ROW_BLOCK
    embed_t = embed.T
    cs = cluster_size.reshape(1, NUM_CODES)

    grid = (n_blocks,)
    out_shapes = (
        jax.ShapeDtypeStruct((n_rows, CODE_DIM), jnp.float32),
        jax.ShapeDtypeStruct((n_rows, 1), jnp.int32),
        jax.ShapeDtypeStruct((1, 8), jnp.float32),
        jax.ShapeDtypeStruct((1, 1), jnp.int32),
    )
    zq, codes, scal, active = pl.pallas_call(
        functools.partial(_vq_kernel, n_rows=n_rows, n_blocks=n_blocks),
        grid=grid,
        in_specs=[
            pl.BlockSpec((ROW_BLOCK, CODE_DIM), lambda i: (i, 0)),
            pl.BlockSpec((NUM_CODES, CODE_DIM), lambda i: (0, 0)),
            pl.BlockSpec((CODE_DIM, NUM_CODES), lambda i: (0, 0)),
            pl.BlockSpec((1, NUM_CODES), lambda i: (0, 0)),
        ],
        out_specs=(
            pl.BlockSpec((ROW_BLOCK, CODE_DIM), lambda i: (i, 0)),
            pl.BlockSpec((ROW_BLOCK, 1), lambda i: (i, 0)),
            pl.BlockSpec((1, 8), lambda i: (0, 0)),
            pl.BlockSpec((1, 1), lambda i: (0, 0)),
        ),
        out_shape=out_shapes,
        scratch_shapes=[
            pltpu.VMEM((1, NUM_CODES), jnp.float32),
            pltpu.VMEM((1, NUM_CODES), jnp.float32),
            pltpu.VMEM((1, NUM_CODES), jnp.float32),
            pltpu.SMEM((1, 1), jnp.float32),
        ],
    )(2.0 * flat_z, embed, embed_t, cs)

    z_q_out = zq.reshape(orig_shape)
    codes_out = codes.reshape(orig_shape[:-1])
    return (z_q_out, codes_out,
            scal[0, 0], scal[0, 1], scal[0, 2], scal[0, 3], scal[0, 4],
            active[0, 0])
